# Initial kernel scaffold; baseline (speedup 1.0000x reference)
import jax
import jax.numpy as jnp
from jax.experimental import pallas as pl


def kernel(x, pos_table):
    raise NotImplementedError("write your pallas kernel here")


# speedup vs baseline: 5.3045x; 5.3045x over previous
import jax
import jax.numpy as jnp
from jax.experimental import pallas as pl

B = 4
SEQ = 2048
D = 768
BS = 512


def _copy_body(tab_ref, out_ref):
    out_ref[...] = jnp.broadcast_to(tab_ref[...][None], (B, BS, D))


def kernel(x, pos_table):
    del x
    grid = (SEQ // BS,)
    return pl.pallas_call(
        _copy_body,
        grid=grid,
        in_specs=[pl.BlockSpec((BS, D), lambda i: (i, 0))],
        out_specs=pl.BlockSpec((B, BS, D), lambda i: (0, i, 0)),
        out_shape=jax.ShapeDtypeStruct((B, SEQ, D), jnp.float32),
    )(pos_table[:SEQ])
